# baseline (speedup 1.0000x reference)
import jax
import jax.numpy as jnp
from jax.experimental import pallas as pl

_NCLS = 11
_BB = 256


def _onehot_kernel(obj_ref, out_ref):
    obj = obj_ref[...]
    bb, hw = obj.shape
    cls = jax.lax.broadcasted_iota(jnp.int32, (bb, _NCLS, hw), 1)
    out_ref[...] = (obj[:, None, :] == cls).astype(jnp.float32)


def kernel(x):
    b, h, w, _ = x.shape
    hw = h * w
    obj = x[..., 0].astype(jnp.int32).reshape(b, hw)
    out = pl.pallas_call(
        _onehot_kernel,
        grid=(b // _BB,),
        in_specs=[pl.BlockSpec((_BB, hw), lambda i: (i, 0))],
        out_specs=pl.BlockSpec((_BB, _NCLS, hw), lambda i: (i, 0, 0)),
        out_shape=jax.ShapeDtypeStruct((b, _NCLS, hw), jnp.float32),
    )(obj)
    return out.reshape(b, _NCLS, h, w)


# speedup vs baseline: 7.1646x; 7.1646x over previous
import jax
import jax.numpy as jnp
from jax.experimental import pallas as pl

_NCLS = 11
_BB = 512


def _onehot_kernel(obj_ref, out_ref):
    obj = obj_ref[:, 0]
    h, w, bb = obj.shape
    cls = jax.lax.broadcasted_iota(jnp.int32, (_NCLS, h, w, bb), 0)
    out_ref[...] = (obj[None] == cls).astype(jnp.float32)


def kernel(x):
    b, h, w, _c = x.shape
    xt = jnp.transpose(x, (1, 3, 2, 0))
    out_t = pl.pallas_call(
        _onehot_kernel,
        grid=(b // _BB,),
        in_specs=[pl.BlockSpec((h, 1, w, _BB), lambda i: (0, 0, 0, i))],
        out_specs=pl.BlockSpec((_NCLS, h, w, _BB), lambda i: (0, 0, 0, i)),
        out_shape=jax.ShapeDtypeStruct((_NCLS, h, w, b), jnp.float32),
    )(xt)
    return jnp.transpose(out_t, (3, 0, 1, 2))
